# baseline (speedup 1.0000x reference)
import jax
import jax.numpy as jnp
from jax.experimental import pallas as pl


def _copy_body(emb_ref, out_ref):
    out_ref[...] = emb_ref[...]


def kernel(emb, t):
    del t
    n, d = emb.shape
    block = 512
    return pl.pallas_call(
        _copy_body,
        grid=(n // block,),
        in_specs=[pl.BlockSpec((block, d), lambda i: (i, 0))],
        out_specs=pl.BlockSpec((block, d), lambda i: (i, 0)),
        out_shape=jax.ShapeDtypeStruct((n, d), emb.dtype),
    )(emb)
